# Initial kernel scaffold; baseline (speedup 1.0000x reference)
import jax
import jax.numpy as jnp
from jax.experimental import pallas as pl


def kernel(x, edge_index, W1, att_src1, att_dst1, b1, fcW1, fcb1, g1, be1, W2, att_src2, att_dst2, b2, fcW2, fcb2, g2, be2, linW, linb):
    raise NotImplementedError("write your pallas kernel here")


# speedup vs baseline: 28.8533x; 28.8533x over previous
import functools

import jax
import jax.numpy as jnp
from jax import lax
from jax.experimental import pallas as pl
from jax.experimental.pallas import tpu as pltpu
from jax.experimental.pallas import tpu_sc as plsc

N = 10000
E = 320000
D = 128
NP = 10240
NW = 32
EB = 64
NB = -(-(E + N) // (NW * EB))
EPW = NB * EB
ET = NW * EPW
RPT = NP // 16


def _sc_gat(heads):
  mesh = plsc.VectorSubcoreMesh(core_axis_name="c", subcore_axis_name="s")

  @functools.partial(
      pl.kernel,
      out_type=(jax.ShapeDtypeStruct((2, NP, D), jnp.float32),
                jax.ShapeDtypeStruct((2, NP, 16), jnp.float32)),
      mesh=mesh,
      compiler_params=pltpu.CompilerParams(use_tc_tiling_on_sc=False),
      scratch_types=[
          pltpu.VMEM((EB,), jnp.int32),
          pltpu.VMEM((EB,), jnp.int32),
          pltpu.VMEM((EB, 16), jnp.float32),
          pltpu.VMEM((EB, 16), jnp.float32),
          pltpu.VMEM((EB, 16), jnp.float32),
          pltpu.VMEM((EB, D), jnp.float32),
          pltpu.VMEM_SHARED((NP, D), jnp.float32),
          pltpu.VMEM_SHARED((NP, 16), jnp.float32),
      ],
  )
  def k(h_hbm, asrc_hbm, adst_hbm, src_hbm, dst_hbm, accout, denout,
        srcb, dstb, arows, brows, pbuf, hrows, acc, den):
    c = lax.axis_index("c")
    s = lax.axis_index("s")
    wid = s * 2 + c

    def zrow(r, carry):
      for j in range(D // 16):
        hrows[r, pl.ds(j * 16, 16)] = jnp.zeros((16,), jnp.float32)
      pbuf[r] = jnp.zeros((16,), jnp.float32)
      return carry
    lax.fori_loop(0, EB, zrow, 0)
    rz = s * RPT
    for t in range(RPT // EB):
      pltpu.sync_copy(hrows, acc.at[pl.ds(rz + t * EB, EB)])
      pltpu.sync_copy(pbuf, den.at[pl.ds(rz + t * EB, EB)])
    plsc.subcore_barrier()

    def eblock(b, carry):
      off = pl.multiple_of(b * EB, EB)
      pltpu.sync_copy(src_hbm.at[wid, pl.ds(off, EB)], srcb)
      pltpu.sync_copy(dst_hbm.at[wid, pl.ds(off, EB)], dstb)
      pltpu.sync_copy(asrc_hbm.at[srcb], arows)
      pltpu.sync_copy(adst_hbm.at[dstb], brows)
      pltpu.sync_copy(h_hbm.at[srcb], hrows)

      def edge(e, ecarry):
        t = arows[e] + brows[e]
        p = jnp.exp(jnp.maximum(t, t * 0.2))
        pbuf[e] = p
        for h8 in range(D // 16):
          ps = p[h8 if heads == 8 else 0]
          hrows[e, pl.ds(h8 * 16, 16)] = hrows[e, pl.ds(h8 * 16, 16)] * ps
        return ecarry
      lax.fori_loop(0, EB, edge, 0)

      pltpu.sync_copy(pbuf, den.at[dstb], add=True)
      pltpu.sync_copy(hrows, acc.at[dstb], add=True)
      return carry
    lax.fori_loop(0, NB, eblock, 0)
    plsc.subcore_barrier()

    ro = s * RPT
    pltpu.sync_copy(acc.at[pl.ds(ro, RPT)], accout.at[c, pl.ds(ro, RPT)])
    pltpu.sync_copy(den.at[pl.ds(ro, RPT)], denout.at[c, pl.ds(ro, RPT)])

  return k


def _head_sum_mat():
  col = lax.broadcasted_iota(jnp.int32, (D, 16), 0) // 16
  hsel = lax.broadcasted_iota(jnp.int32, (D, 16), 1)
  return (col == hsel).astype(jnp.float32)


def _tc_pre_body(x_ref, w_ref, as_ref, ad_ref, h_ref, s_ref, d_ref):
  h = jnp.dot(x_ref[...], w_ref[...], preferred_element_type=jnp.float32)
  h_ref[...] = h
  S = _head_sum_mat()
  s_ref[...] = jnp.dot(h * as_ref[...], S, preferred_element_type=jnp.float32)
  d_ref[...] = jnp.dot(h * ad_ref[...], S, preferred_element_type=jnp.float32)


def _tc_pre(x, W1, a1s, a1d):
  return pl.pallas_call(
      _tc_pre_body,
      out_shape=(jax.ShapeDtypeStruct((N, D), jnp.float32),
                 jax.ShapeDtypeStruct((N, 16), jnp.float32),
                 jax.ShapeDtypeStruct((N, 16), jnp.float32)),
  )(x, W1, a1s, a1d)


def _norm_gat(accp, denp, brow, rep):
  acc = accp[0, :N, :] + accp[1, :N, :]
  den = denp[0, :N, :] + denp[1, :N, :]
  denrep = jnp.dot(den, rep, preferred_element_type=jnp.float32)
  return acc / (denrep + 1e-16) + brow


def _bn(y, grow, berow):
  m = jnp.mean(y, axis=0, keepdims=True)
  v = jnp.mean((y - m) ** 2, axis=0, keepdims=True)
  return (y - m) * lax.rsqrt(v + 1e-5) * grow + berow


def _tc_mid_body(accp_ref, denp_ref, b_ref, fw_ref, fb_ref, g_ref, be_ref,
                 w2_ref, a2s_ref, a2d_ref, h2_ref, s2_ref, d2_ref):
  hsel = lax.broadcasted_iota(jnp.int32, (16, D), 0)
  col = lax.broadcasted_iota(jnp.int32, (16, D), 1) // 16
  rep = (hsel == col).astype(jnp.float32)
  gat = _norm_gat(accp_ref[...], denp_ref[...], b_ref[...], rep)
  y = jnp.maximum(jnp.dot(gat, fw_ref[...],
                          preferred_element_type=jnp.float32) + fb_ref[...], 0.0)
  ybn = _bn(y, g_ref[...], be_ref[...])
  h2 = jnp.dot(ybn, w2_ref[...], preferred_element_type=jnp.float32)
  h2_ref[...] = h2
  ones = jnp.ones((D, 16), jnp.float32)
  mask0 = (lax.broadcasted_iota(jnp.int32, (1, 16), 1) == 0).astype(jnp.float32)
  s2_ref[...] = jnp.dot(h2 * a2s_ref[...], ones,
                        preferred_element_type=jnp.float32) * mask0
  d2_ref[...] = jnp.dot(h2 * a2d_ref[...], ones,
                        preferred_element_type=jnp.float32) * mask0


def _tc_mid(accp, denp, brow, fw, fbrow, grow, berow, W2, a2s, a2d):
  return pl.pallas_call(
      _tc_mid_body,
      out_shape=(jax.ShapeDtypeStruct((N, D), jnp.float32),
                 jax.ShapeDtypeStruct((N, 16), jnp.float32),
                 jax.ShapeDtypeStruct((N, 16), jnp.float32)),
  )(accp, denp, brow, fw, fbrow, grow, berow, W2, a2s, a2d)


def _tc_fin_body(accp_ref, denp_ref, b_ref, fw_ref, fb_ref, g_ref, be_ref,
                 lw_ref, lb_ref, o_ref):
  rep = (lax.broadcasted_iota(jnp.int32, (16, D), 0) == 0).astype(jnp.float32)
  gat = _norm_gat(accp_ref[...], denp_ref[...], b_ref[...], rep)
  z = jnp.maximum(jnp.dot(gat, fw_ref[...],
                          preferred_element_type=jnp.float32) + fb_ref[...], 0.0)
  zbn = _bn(z, g_ref[...], be_ref[...])
  out = jnp.dot(zbn, lw_ref[...], preferred_element_type=jnp.float32) + lb_ref[...]
  o_ref[...] = jnp.maximum(out, 0.0)


def _tc_fin(accp, denp, brow, fw, fbrow, grow, berow, linW, lbrow):
  return pl.pallas_call(
      _tc_fin_body,
      out_shape=jax.ShapeDtypeStruct((N, D), jnp.float32),
  )(accp, denp, brow, fw, fbrow, grow, berow, linW, lbrow)


_sc_gat8 = _sc_gat(8)
_sc_gat1 = _sc_gat(1)


def kernel(x, edge_index, W1, att_src1, att_dst1, b1, fcW1, fcb1, g1, be1,
           W2, att_src2, att_dst2, b2, fcW2, fcb2, g2, be2, linW, linb):
  loop = jnp.arange(N, dtype=jnp.int32)
  pad = ET - (E + N)
  src_p = jnp.concatenate(
      [edge_index[0].astype(jnp.int32), loop,
       jnp.zeros((pad,), jnp.int32)]).reshape(NW, EPW)
  dst_p = jnp.concatenate(
      [edge_index[1].astype(jnp.int32), loop,
       jnp.full((pad,), N, jnp.int32)]).reshape(NW, EPW)

  row = lambda a: a.reshape(1, D)
  h1, asrc1, adst1 = _tc_pre(x, W1, row(att_src1), row(att_dst1))
  padn = lambda a: jnp.pad(a, ((0, NP - N), (0, 0)))
  accp, denp = _sc_gat8(h1, padn(asrc1), padn(adst1), src_p, dst_p)

  h2, asrc2, adst2 = _tc_mid(accp, denp, row(b1), fcW1, row(fcb1), row(g1),
                             row(be1), W2, row(att_src2), row(att_dst2))
  accp2, denp2 = _sc_gat1(h2, padn(asrc2), padn(adst2), src_p, dst_p)

  return _tc_fin(accp2, denp2, row(b2), fcW2, row(fcb2), row(g2), row(be2),
                 linW, row(linb))
